# baseline (speedup 1.0000x reference)
import functools

import jax
import jax.numpy as jnp
from jax import lax
from jax.experimental import pallas as pl
from jax.experimental.pallas import tpu as pltpu
from jax.experimental.pallas import tpu_sc as plsc

_NC = 2
_NS = 16
_NW = _NC * _NS
_QB = 489
_Q = _QB * 512
_CHUNK = 256


def _quad_body(u0, u1, u2, u3, i0, i1, i2, i3, ou, oi):
    eye = jnp.eye(64, dtype=jnp.float32)

    def t16(x):
        xt = lax.dot_general(x[...], eye, (((0,), (0,)), ((), ())),
                             preferred_element_type=jnp.float32)
        b = lax.bitcast_convert_type(xt.astype(jnp.bfloat16), jnp.uint16)
        return b.astype(jnp.uint32)

    for parts, o in (((u0, u1, u2, u3), ou), ((i0, i1, i2, i3), oi)):
        a0, a1, a2, a3 = (t16(x) for x in parts)
        w01 = lax.bitcast_convert_type((a1 << 16) | a0, jnp.float32)
        w23 = lax.bitcast_convert_type((a3 << 16) | a2, jnp.float32)
        o[:, :64] = w01
        o[:, 64:] = w23


def _quad2(tu, ti):
    D = 64
    ins = []
    specs = []
    nblk = (tu.shape[1] + 511) // 512
    for t in (tu, ti):
        for k in range(4):
            ins.append(t)
            specs.append(
                pl.BlockSpec((D, 512), functools.partial(
                    lambda k, j: (0, jnp.minimum(j + k * _QB, nblk - 1)), k)))
    spec_out = pl.BlockSpec((512, 128), lambda j: (j, 0))
    return pl.pallas_call(
        _quad_body,
        grid=(_QB,),
        in_specs=specs,
        out_specs=[spec_out] * 2,
        out_shape=[jax.ShapeDtypeStruct((_Q, 128), jnp.float32)] * 2,
        compiler_params=pltpu.CompilerParams(
            dimension_semantics=("arbitrary",)),
    )(*ins)


def _gather_mf(uidx, iidx, t_umf, t_imf):
    B = uidx.shape[0]
    D = t_umf.shape[1]
    bw = B // _NW

    mesh = plsc.VectorSubcoreMesh(
        core_axis_name="c", subcore_axis_name="s",
        num_cores=_NC, num_subcores=_NS)

    @functools.partial(
        pl.kernel,
        mesh=mesh,
        out_type=[jax.ShapeDtypeStruct((B, D), jnp.float32)] * 2,
        scratch_types=[
            pltpu.VMEM((bw,), jnp.int32),
            pltpu.VMEM((bw,), jnp.int32),
            pltpu.VMEM((bw, D), jnp.float32),
            pltpu.VMEM((bw, D), jnp.float32),
            pltpu.SemaphoreType.DMA,
            pltpu.SemaphoreType.DMA,
        ],
        compiler_params=pltpu.CompilerParams(use_tc_tiling_on_sc=False),
    )
    def k(uidx_hbm, iidx_hbm, umf_hbm, imf_hbm, out_umf, out_imf,
          uidx_v, iidx_v, buf0, buf1, sem0, sem1):
        wid = lax.axis_index("s") * _NC + lax.axis_index("c")
        base = wid * bw
        pltpu.sync_copy(uidx_hbm.at[pl.ds(base, bw)], uidx_v)
        pltpu.sync_copy(iidx_hbm.at[pl.ds(base, bw)], iidx_v)
        cp0 = pltpu.async_copy(umf_hbm.at[uidx_v], buf0, sem0)
        cp1 = pltpu.async_copy(imf_hbm.at[iidx_v], buf1, sem1)
        cp0.wait()
        pltpu.sync_copy(buf0, out_umf.at[pl.ds(base, bw)])
        cp1.wait()
        pltpu.sync_copy(buf1, out_imf.at[pl.ds(base, bw)])

    return k(uidx, iidx, t_umf, t_imf)


def _gather_mlp(quidx, qiidx, p_umlp, p_imlp):
    B = quidx.shape[0]
    D = p_umlp.shape[1]
    bw = B // _NW
    nchunk = bw // _CHUNK

    mesh = plsc.VectorSubcoreMesh(
        core_axis_name="c", subcore_axis_name="s",
        num_cores=_NC, num_subcores=_NS)

    @functools.partial(
        pl.kernel,
        mesh=mesh,
        out_type=[jax.ShapeDtypeStruct((B, D), jnp.float32)] * 2,
        scratch_types=[
            pltpu.VMEM((bw,), jnp.int32),
            pltpu.VMEM((bw,), jnp.int32),
            pltpu.VMEM((_CHUNK, D), jnp.float32),
            pltpu.VMEM((_CHUNK, D), jnp.float32),
            pltpu.SemaphoreType.DMA,
            pltpu.SemaphoreType.DMA,
        ],
    )
    def k(uidx_hbm, iidx_hbm, umlp_hbm, imlp_hbm, out_umlp, out_imlp,
          uidx_v, iidx_v, buf0, buf1, sem0, sem1):
        wid = lax.axis_index("s") * _NC + lax.axis_index("c")
        base = wid * bw
        pltpu.sync_copy(uidx_hbm.at[pl.ds(base, bw)], uidx_v)
        pltpu.sync_copy(iidx_hbm.at[pl.ds(base, bw)], iidx_v)

        def chunk_body(c, _):
            cb = c * _CHUNK
            iu = uidx_v.at[pl.ds(cb, _CHUNK)]
            ii = iidx_v.at[pl.ds(cb, _CHUNK)]
            od = pl.ds(base + cb, _CHUNK)
            cp0 = pltpu.async_copy(umlp_hbm.at[iu], buf0, sem0)
            cp1 = pltpu.async_copy(imlp_hbm.at[ii], buf1, sem1)
            cp0.wait()
            pltpu.sync_copy(buf0, out_umlp.at[od])
            cp1.wait()
            pltpu.sync_copy(buf1, out_imlp.at[od])
            return _

        lax.fori_loop(0, nchunk, chunk_body, None)

    return k(quidx, qiidx, p_umlp, p_imlp)


def _tower_body(gumlp, gimlp, umf, imf, uq, iq, w1a, w1b, b1, w2, b2,
                w3, b3, womlp, womf, bo, out):
    def unpack(g, q):
        w = lax.bitcast_convert_type(g[...], jnp.uint32)
        sel = jnp.where(q < 2, w[:, 0:64], w[:, 64:128])
        hb = jnp.where((q & 1) == 1, sel >> 16, sel & 0xFFFF)
        bf = lax.bitcast_convert_type(hb.astype(jnp.uint16), jnp.bfloat16)
        return bf.astype(jnp.float32)

    umlp = unpack(gumlp, uq[...])
    imlp = unpack(gimlp, iq[...])
    h = (jnp.dot(umlp, w1a[...], preferred_element_type=jnp.float32)
         + jnp.dot(imlp, w1b[...], preferred_element_type=jnp.float32)
         + b1[...])
    h = jnp.maximum(h, 0.0)
    h = jnp.maximum(
        jnp.dot(h, w2[...], preferred_element_type=jnp.float32) + b2[...], 0.0)
    h = jnp.maximum(
        jnp.dot(h, w3[...], preferred_element_type=jnp.float32) + b3[...], 0.0)
    mf = umf[...] * imf[...]
    logit = (jnp.sum(h * womlp[...], axis=-1, keepdims=True)
             + jnp.sum(mf * womf[...], axis=-1, keepdims=True)
             + bo[...])
    out[...] = jax.nn.sigmoid(logit)


def _tower(gumlp, gimlp, umf, imf, uq, iq,
           W1, b1, W2, b2, W3, b3, W_out, b_out):
    B = umf.shape[0]
    D = 64
    grid = 8
    bm = B // grid
    w1a, w1b = W1[:D], W1[D:]
    womlp = W_out[:16, 0].reshape(1, 16)
    womf = W_out[16:, 0].reshape(1, D)

    full = lambda i: (0, 0)
    row = lambda i: (i, 0)
    out = pl.pallas_call(
        _tower_body,
        grid=(grid,),
        in_specs=[
            pl.BlockSpec((bm, 2 * D), row),
            pl.BlockSpec((bm, 2 * D), row),
            pl.BlockSpec((bm, D), row),
            pl.BlockSpec((bm, D), row),
            pl.BlockSpec((bm, 1), row),
            pl.BlockSpec((bm, 1), row),
            pl.BlockSpec((D, 64), full),
            pl.BlockSpec((D, 64), full),
            pl.BlockSpec((1, 64), full),
            pl.BlockSpec((64, 32), full),
            pl.BlockSpec((1, 32), full),
            pl.BlockSpec((32, 16), full),
            pl.BlockSpec((1, 16), full),
            pl.BlockSpec((1, 16), full),
            pl.BlockSpec((1, D), full),
            pl.BlockSpec((1, 1), full),
        ],
        out_specs=pl.BlockSpec((bm, 1), row),
        out_shape=jax.ShapeDtypeStruct((B, 1), jnp.float32),
        compiler_params=pltpu.CompilerParams(
            dimension_semantics=("arbitrary",)),
    )(gumlp, gimlp, umf, imf, uq, iq, w1a, w1b, b1.reshape(1, -1), W2,
      b2.reshape(1, -1), W3, b3.reshape(1, -1), womlp, womf,
      b_out.reshape(1, 1))
    return out[:, 0]


def kernel(user_indices, item_indices, emb_user_mf, emb_item_mf,
           emb_user_mlp, emb_item_mlp, W1, b1, W2, b2, W3, b3, W_out, b_out):
    ui = user_indices.astype(jnp.int32)
    ii = item_indices.astype(jnp.int32)
    p_umlp, p_imlp = _quad2(emb_user_mlp.T, emb_item_mlp.T)
    uq = ui // _Q
    iq = ii // _Q
    umf, imf = _gather_mf(ui, ii, emb_user_mf, emb_item_mf)
    gumlp, gimlp = _gather_mlp(ui - uq * _Q, ii - iq * _Q, p_umlp, p_imlp)
    return _tower(gumlp, gimlp, umf, imf,
                  uq.reshape(-1, 1), iq.reshape(-1, 1),
                  W1, b1, W2, b2, W3, b3, W_out, b_out)


# speedup vs baseline: 1.8409x; 1.8409x over previous
import functools

import jax
import jax.numpy as jnp
from jax import lax
from jax.experimental import pallas as pl
from jax.experimental.pallas import tpu as pltpu
from jax.experimental.pallas import tpu_sc as plsc

_NC = 2
_NS = 16
_NW = _NC * _NS
_QB = 489
_Q = _QB * 512
_CHUNK = 256


def _quad_body(*refs):
    ins, outs = refs[:16], refs[16:]
    eye = jnp.eye(64, dtype=jnp.float32)

    def t16(x):
        xt = lax.dot_general(x[...], eye, (((0,), (0,)), ((), ())),
                             preferred_element_type=jnp.float32)
        b = lax.bitcast_convert_type(xt.astype(jnp.bfloat16), jnp.uint16)
        return b.astype(jnp.uint32)

    for i, o in enumerate(outs):
        a0, a1, a2, a3 = (t16(x) for x in ins[4 * i:4 * i + 4])
        o[:, :64] = lax.bitcast_convert_type((a1 << 16) | a0, jnp.float32)
        o[:, 64:] = lax.bitcast_convert_type((a3 << 16) | a2, jnp.float32)


def _quad4(t0, t1, t2, t3):
    D = 64
    nblk = (t0.shape[1] + 511) // 512
    ins = []
    specs = []
    for t in (t0, t1, t2, t3):
        for k in range(4):
            ins.append(t)
            specs.append(
                pl.BlockSpec((D, 512), functools.partial(
                    lambda k, j: (0, jnp.minimum(j + k * _QB, nblk - 1)), k)))
    spec_out = pl.BlockSpec((512, 128), lambda j: (j, 0))
    return pl.pallas_call(
        _quad_body,
        grid=(_QB,),
        in_specs=specs,
        out_specs=[spec_out] * 4,
        out_shape=[jax.ShapeDtypeStruct((_Q, 128), jnp.float32)] * 4,
        compiler_params=pltpu.CompilerParams(
            dimension_semantics=("arbitrary",)),
    )(*ins)


def _gather4(quidx, qiidx, p_umf, p_imf, p_umlp, p_imlp):
    B = quidx.shape[0]
    D = p_umf.shape[1]
    bw = B // _NW
    nchunk = bw // _CHUNK

    mesh = plsc.VectorSubcoreMesh(
        core_axis_name="c", subcore_axis_name="s",
        num_cores=_NC, num_subcores=_NS)

    @functools.partial(
        pl.kernel,
        mesh=mesh,
        out_type=[jax.ShapeDtypeStruct((B, D), jnp.float32)] * 4,
        scratch_types=[
            pltpu.VMEM((bw,), jnp.int32),
            pltpu.VMEM((bw,), jnp.int32),
            pltpu.VMEM((_CHUNK, D), jnp.float32),
            pltpu.VMEM((_CHUNK, D), jnp.float32),
            pltpu.SemaphoreType.DMA,
            pltpu.SemaphoreType.DMA,
        ],
    )
    def k(uidx_hbm, iidx_hbm, umf_hbm, imf_hbm, umlp_hbm, imlp_hbm,
          out_umf, out_imf, out_umlp, out_imlp,
          uidx_v, iidx_v, buf0, buf1, sem0, sem1):
        wid = lax.axis_index("s") * _NC + lax.axis_index("c")
        base = wid * bw
        pltpu.sync_copy(uidx_hbm.at[pl.ds(base, bw)], uidx_v)
        pltpu.sync_copy(iidx_hbm.at[pl.ds(base, bw)], iidx_v)

        def chunk_body(c, _):
            cb = c * _CHUNK
            iu = uidx_v.at[pl.ds(cb, _CHUNK)]
            ii = iidx_v.at[pl.ds(cb, _CHUNK)]
            od = pl.ds(base + cb, _CHUNK)
            cp0 = pltpu.async_copy(umf_hbm.at[iu], buf0, sem0)
            cp1 = pltpu.async_copy(imf_hbm.at[ii], buf1, sem1)
            cp0.wait()
            pltpu.sync_copy(buf0, out_umf.at[od])
            cp0 = pltpu.async_copy(umlp_hbm.at[iu], buf0, sem0)
            cp1.wait()
            pltpu.sync_copy(buf1, out_imf.at[od])
            cp1 = pltpu.async_copy(imlp_hbm.at[ii], buf1, sem1)
            cp0.wait()
            pltpu.sync_copy(buf0, out_umlp.at[od])
            cp1.wait()
            pltpu.sync_copy(buf1, out_imlp.at[od])
            return _

        lax.fori_loop(0, nchunk, chunk_body, None)

    return k(quidx, qiidx, p_umf, p_imf, p_umlp, p_imlp)


def _tower_body(gumlp, gimlp, gumf, gimf, uq, iq, w1a, w1b, b1, w2, b2,
                w3, b3, womlp, womf, bo, out):
    def unpack(g, q):
        w = lax.bitcast_convert_type(g[...], jnp.uint32)
        sel = jnp.where(q < 2, w[:, 0:64], w[:, 64:128])
        hb = jnp.where((q & 1) == 1, sel >> 16, sel & 0xFFFF)
        bf = lax.bitcast_convert_type(hb.astype(jnp.uint16), jnp.bfloat16)
        return bf.astype(jnp.float32)

    umlp = unpack(gumlp, uq[...])
    imlp = unpack(gimlp, iq[...])
    umf = unpack(gumf, uq[...])
    imf = unpack(gimf, iq[...])
    h = (jnp.dot(umlp, w1a[...], preferred_element_type=jnp.float32)
         + jnp.dot(imlp, w1b[...], preferred_element_type=jnp.float32)
         + b1[...])
    h = jnp.maximum(h, 0.0)
    h = jnp.maximum(
        jnp.dot(h, w2[...], preferred_element_type=jnp.float32) + b2[...], 0.0)
    h = jnp.maximum(
        jnp.dot(h, w3[...], preferred_element_type=jnp.float32) + b3[...], 0.0)
    mf = umf * imf
    logit = (jnp.sum(h * womlp[...], axis=-1, keepdims=True)
             + jnp.sum(mf * womf[...], axis=-1, keepdims=True)
             + bo[...])
    out[...] = jax.nn.sigmoid(logit)


def _tower(gumlp, gimlp, gumf, gimf, uq, iq,
           W1, b1, W2, b2, W3, b3, W_out, b_out):
    B = gumf.shape[0]
    D = 64
    grid = 8
    bm = B // grid
    w1a, w1b = W1[:D], W1[D:]
    womlp = W_out[:16, 0].reshape(1, 16)
    womf = W_out[16:, 0].reshape(1, D)

    full = lambda i: (0, 0)
    row = lambda i: (i, 0)
    out = pl.pallas_call(
        _tower_body,
        grid=(grid,),
        in_specs=[
            pl.BlockSpec((bm, 2 * D), row),
            pl.BlockSpec((bm, 2 * D), row),
            pl.BlockSpec((bm, 2 * D), row),
            pl.BlockSpec((bm, 2 * D), row),
            pl.BlockSpec((bm, 1), row),
            pl.BlockSpec((bm, 1), row),
            pl.BlockSpec((D, 64), full),
            pl.BlockSpec((D, 64), full),
            pl.BlockSpec((1, 64), full),
            pl.BlockSpec((64, 32), full),
            pl.BlockSpec((1, 32), full),
            pl.BlockSpec((32, 16), full),
            pl.BlockSpec((1, 16), full),
            pl.BlockSpec((1, 16), full),
            pl.BlockSpec((1, D), full),
            pl.BlockSpec((1, 1), full),
        ],
        out_specs=pl.BlockSpec((bm, 1), row),
        out_shape=jax.ShapeDtypeStruct((B, 1), jnp.float32),
        compiler_params=pltpu.CompilerParams(
            dimension_semantics=("arbitrary",)),
    )(gumlp, gimlp, gumf, gimf, uq, iq, w1a, w1b, b1.reshape(1, -1), W2,
      b2.reshape(1, -1), W3, b3.reshape(1, -1), womlp, womf,
      b_out.reshape(1, 1))
    return out[:, 0]


def kernel(user_indices, item_indices, emb_user_mf, emb_item_mf,
           emb_user_mlp, emb_item_mlp, W1, b1, W2, b2, W3, b3, W_out, b_out):
    ui = user_indices.astype(jnp.int32)
    ii = item_indices.astype(jnp.int32)
    p_umf, p_imf, p_umlp, p_imlp = _quad4(
        emb_user_mf.T, emb_item_mf.T, emb_user_mlp.T, emb_item_mlp.T)
    uq = ui // _Q
    iq = ii // _Q
    gumf, gimf, gumlp, gimlp = _gather4(
        ui - uq * _Q, ii - iq * _Q, p_umf, p_imf, p_umlp, p_imlp)
    return _tower(gumlp, gimlp, gumf, gimf,
                  uq.reshape(-1, 1), iq.reshape(-1, 1),
                  W1, b1, W2, b2, W3, b3, W_out, b_out)


# speedup vs baseline: 2.1086x; 1.1454x over previous
import functools

import jax
import jax.numpy as jnp
from jax import lax
from jax.experimental import pallas as pl
from jax.experimental.pallas import tpu as pltpu
from jax.experimental.pallas import tpu_sc as plsc

_NC = 2
_NS = 16
_NW = _NC * _NS
_QW = 1024
_QB = 245
_Q = _QB * _QW
_CHUNK = 256


def _quad_body(*refs):
    ins, outs = refs[:16], refs[16:]
    eye = jnp.eye(64, dtype=jnp.float32)

    def t16(x):
        xt = lax.dot_general(x[...], eye, (((0,), (0,)), ((), ())),
                             preferred_element_type=jnp.float32)
        b = lax.bitcast_convert_type(xt.astype(jnp.bfloat16), jnp.uint16)
        return b.astype(jnp.uint32)

    for i, o in enumerate(outs):
        a0, a1, a2, a3 = (t16(x) for x in ins[4 * i:4 * i + 4])
        o[:, :64] = lax.bitcast_convert_type((a1 << 16) | a0, jnp.float32)
        o[:, 64:] = lax.bitcast_convert_type((a3 << 16) | a2, jnp.float32)


def _quad4(t0, t1, t2, t3):
    D = 64
    nblk = (t0.shape[1] + _QW - 1) // _QW
    ins = []
    specs = []
    for t in (t0, t1, t2, t3):
        for k in range(4):
            ins.append(t)
            specs.append(
                pl.BlockSpec((D, _QW), functools.partial(
                    lambda k, j: (0, jnp.minimum(j + k * _QB, nblk - 1)), k)))
    spec_out = pl.BlockSpec((_QW, 128), lambda j: (j, 0))
    return pl.pallas_call(
        _quad_body,
        grid=(_QB,),
        in_specs=specs,
        out_specs=[spec_out] * 4,
        out_shape=[jax.ShapeDtypeStruct((_Q, 128), jnp.float32)] * 4,
        compiler_params=pltpu.CompilerParams(
            dimension_semantics=("arbitrary",)),
    )(*ins)


def _gather4(quidx, qiidx, p_umf, p_imf, p_umlp, p_imlp):
    B = quidx.shape[0]
    D = p_umf.shape[1]
    bw = B // _NW
    nchunk = bw // _CHUNK

    mesh = plsc.VectorSubcoreMesh(
        core_axis_name="c", subcore_axis_name="s",
        num_cores=_NC, num_subcores=_NS)

    @functools.partial(
        pl.kernel,
        mesh=mesh,
        out_type=[jax.ShapeDtypeStruct((B, D), jnp.float32)] * 4,
        scratch_types=[
            pltpu.VMEM((bw,), jnp.int32),
            pltpu.VMEM((bw,), jnp.int32),
            pltpu.VMEM((_CHUNK, D), jnp.float32),
            pltpu.VMEM((_CHUNK, D), jnp.float32),
            pltpu.SemaphoreType.DMA,
            pltpu.SemaphoreType.DMA,
        ],
    )
    def k(uidx_hbm, iidx_hbm, umf_hbm, imf_hbm, umlp_hbm, imlp_hbm,
          out_umf, out_imf, out_umlp, out_imlp,
          uidx_v, iidx_v, buf0, buf1, sem0, sem1):
        wid = lax.axis_index("s") * _NC + lax.axis_index("c")
        base = wid * bw
        pltpu.sync_copy(uidx_hbm.at[pl.ds(base, bw)], uidx_v)
        pltpu.sync_copy(iidx_hbm.at[pl.ds(base, bw)], iidx_v)

        def chunk_body(c, _):
            cb = c * _CHUNK
            iu = uidx_v.at[pl.ds(cb, _CHUNK)]
            ii = iidx_v.at[pl.ds(cb, _CHUNK)]
            od = pl.ds(base + cb, _CHUNK)
            cp0 = pltpu.async_copy(umf_hbm.at[iu], buf0, sem0)
            cp1 = pltpu.async_copy(imf_hbm.at[ii], buf1, sem1)
            cp0.wait()
            pltpu.sync_copy(buf0, out_umf.at[od])
            cp0 = pltpu.async_copy(umlp_hbm.at[iu], buf0, sem0)
            cp1.wait()
            pltpu.sync_copy(buf1, out_imf.at[od])
            cp1 = pltpu.async_copy(imlp_hbm.at[ii], buf1, sem1)
            cp0.wait()
            pltpu.sync_copy(buf0, out_umlp.at[od])
            cp1.wait()
            pltpu.sync_copy(buf1, out_imlp.at[od])
            return _

        lax.fori_loop(0, nchunk, chunk_body, None)

    return k(quidx, qiidx, p_umf, p_imf, p_umlp, p_imlp)


def _tower_body(gumlp, gimlp, gumf, gimf, uq, iq, w1a, w1b, b1, w2, b2,
                w3, b3, womlp, womf, bo, out):
    def unpack(g, q):
        w = lax.bitcast_convert_type(g[...], jnp.uint32)
        sel = jnp.where(q < 2, w[:, 0:64], w[:, 64:128])
        hb = jnp.where((q & 1) == 1, sel >> 16, sel & 0xFFFF)
        bf = lax.bitcast_convert_type(hb.astype(jnp.uint16), jnp.bfloat16)
        return bf.astype(jnp.float32)

    umlp = unpack(gumlp, uq[...])
    imlp = unpack(gimlp, iq[...])
    umf = unpack(gumf, uq[...])
    imf = unpack(gimf, iq[...])
    h = (jnp.dot(umlp, w1a[...], preferred_element_type=jnp.float32)
         + jnp.dot(imlp, w1b[...], preferred_element_type=jnp.float32)
         + b1[...])
    h = jnp.maximum(h, 0.0)
    h = jnp.maximum(
        jnp.dot(h, w2[...], preferred_element_type=jnp.float32) + b2[...], 0.0)
    h = jnp.maximum(
        jnp.dot(h, w3[...], preferred_element_type=jnp.float32) + b3[...], 0.0)
    mf = umf * imf
    logit = (jnp.sum(h * womlp[...], axis=-1, keepdims=True)
             + jnp.sum(mf * womf[...], axis=-1, keepdims=True)
             + bo[...])
    out[...] = jax.nn.sigmoid(logit)


def _tower(gumlp, gimlp, gumf, gimf, uq, iq,
           W1, b1, W2, b2, W3, b3, W_out, b_out):
    B = gumf.shape[0]
    D = 64
    grid = 8
    bm = B // grid
    w1a, w1b = W1[:D], W1[D:]
    womlp = W_out[:16, 0].reshape(1, 16)
    womf = W_out[16:, 0].reshape(1, D)

    full = lambda i: (0, 0)
    row = lambda i: (i, 0)
    out = pl.pallas_call(
        _tower_body,
        grid=(grid,),
        in_specs=[
            pl.BlockSpec((bm, 2 * D), row),
            pl.BlockSpec((bm, 2 * D), row),
            pl.BlockSpec((bm, 2 * D), row),
            pl.BlockSpec((bm, 2 * D), row),
            pl.BlockSpec((bm, 1), row),
            pl.BlockSpec((bm, 1), row),
            pl.BlockSpec((D, 64), full),
            pl.BlockSpec((D, 64), full),
            pl.BlockSpec((1, 64), full),
            pl.BlockSpec((64, 32), full),
            pl.BlockSpec((1, 32), full),
            pl.BlockSpec((32, 16), full),
            pl.BlockSpec((1, 16), full),
            pl.BlockSpec((1, 16), full),
            pl.BlockSpec((1, D), full),
            pl.BlockSpec((1, 1), full),
        ],
        out_specs=pl.BlockSpec((bm, 1), row),
        out_shape=jax.ShapeDtypeStruct((B, 1), jnp.float32),
        compiler_params=pltpu.CompilerParams(
            dimension_semantics=("arbitrary",)),
    )(gumlp, gimlp, gumf, gimf, uq, iq, w1a, w1b, b1.reshape(1, -1), W2,
      b2.reshape(1, -1), W3, b3.reshape(1, -1), womlp, womf,
      b_out.reshape(1, 1))
    return out[:, 0]


def kernel(user_indices, item_indices, emb_user_mf, emb_item_mf,
           emb_user_mlp, emb_item_mlp, W1, b1, W2, b2, W3, b3, W_out, b_out):
    ui = user_indices.astype(jnp.int32)
    ii = item_indices.astype(jnp.int32)
    p_umf, p_imf, p_umlp, p_imlp = _quad4(
        emb_user_mf.T, emb_item_mf.T, emb_user_mlp.T, emb_item_mlp.T)
    uq = ui // _Q
    iq = ii // _Q
    gumf, gimf, gumlp, gimlp = _gather4(
        ui - uq * _Q, ii - iq * _Q, p_umf, p_imf, p_umlp, p_imlp)
    return _tower(gumlp, gimlp, gumf, gimf,
                  uq.reshape(-1, 1), iq.reshape(-1, 1),
                  W1, b1, W2, b2, W3, b3, W_out, b_out)


# speedup vs baseline: 2.2017x; 1.0441x over previous
import functools

import jax
import jax.numpy as jnp
from jax import lax
from jax.experimental import pallas as pl
from jax.experimental.pallas import tpu as pltpu
from jax.experimental.pallas import tpu_sc as plsc

_NC = 2
_NS = 16
_NW = _NC * _NS
_QW = 2048
_QB = 123
_Q = _QB * _QW
_CHUNK = 256


def _quad_body(*refs):
    ins, outs = refs[:16], refs[16:]
    eye = jnp.eye(64, dtype=jnp.float32)

    def t16(x):
        xt = lax.dot_general(x[...], eye, (((0,), (0,)), ((), ())),
                             preferred_element_type=jnp.float32)
        b = lax.bitcast_convert_type(xt.astype(jnp.bfloat16), jnp.uint16)
        return b.astype(jnp.uint32)

    for i, o in enumerate(outs):
        a0, a1, a2, a3 = (t16(x) for x in ins[4 * i:4 * i + 4])
        o[:, :64] = lax.bitcast_convert_type((a1 << 16) | a0, jnp.float32)
        o[:, 64:] = lax.bitcast_convert_type((a3 << 16) | a2, jnp.float32)


def _quad4(t0, t1, t2, t3):
    D = 64
    nblk = (t0.shape[1] + _QW - 1) // _QW
    ins = []
    specs = []
    for t in (t0, t1, t2, t3):
        for k in range(4):
            ins.append(t)
            specs.append(
                pl.BlockSpec((D, _QW), functools.partial(
                    lambda k, j: (0, jnp.minimum(j + k * _QB, nblk - 1)), k)))
    spec_out = pl.BlockSpec((_QW, 128), lambda j: (j, 0))
    return pl.pallas_call(
        _quad_body,
        grid=(_QB,),
        in_specs=specs,
        out_specs=[spec_out] * 4,
        out_shape=[jax.ShapeDtypeStruct((_Q, 128), jnp.float32)] * 4,
        compiler_params=pltpu.CompilerParams(
            dimension_semantics=("arbitrary",)),
    )(*ins)


def _gather4(quidx, qiidx, p_umf, p_imf, p_umlp, p_imlp):
    B = quidx.shape[0]
    D = p_umf.shape[1]
    bw = B // _NW
    nchunk = bw // _CHUNK

    mesh = plsc.VectorSubcoreMesh(
        core_axis_name="c", subcore_axis_name="s",
        num_cores=_NC, num_subcores=_NS)

    @functools.partial(
        pl.kernel,
        mesh=mesh,
        out_type=[jax.ShapeDtypeStruct((B, D), jnp.float32)] * 4,
        scratch_types=[
            pltpu.VMEM((bw,), jnp.int32),
            pltpu.VMEM((bw,), jnp.int32),
            pltpu.VMEM((_CHUNK, D), jnp.float32),
            pltpu.VMEM((_CHUNK, D), jnp.float32),
            pltpu.SemaphoreType.DMA,
            pltpu.SemaphoreType.DMA,
        ],
    )
    def k(uidx_hbm, iidx_hbm, umf_hbm, imf_hbm, umlp_hbm, imlp_hbm,
          out_umf, out_imf, out_umlp, out_imlp,
          uidx_v, iidx_v, buf0, buf1, sem0, sem1):
        wid = lax.axis_index("s") * _NC + lax.axis_index("c")
        base = wid * bw
        pltpu.sync_copy(uidx_hbm.at[pl.ds(base, bw)], uidx_v)
        pltpu.sync_copy(iidx_hbm.at[pl.ds(base, bw)], iidx_v)

        def chunk_body(c, _):
            cb = c * _CHUNK
            iu = uidx_v.at[pl.ds(cb, _CHUNK)]
            ii = iidx_v.at[pl.ds(cb, _CHUNK)]
            od = pl.ds(base + cb, _CHUNK)
            cp0 = pltpu.async_copy(umf_hbm.at[iu], buf0, sem0)
            cp1 = pltpu.async_copy(imf_hbm.at[ii], buf1, sem1)
            cp0.wait()
            pltpu.sync_copy(buf0, out_umf.at[od])
            cp0 = pltpu.async_copy(umlp_hbm.at[iu], buf0, sem0)
            cp1.wait()
            pltpu.sync_copy(buf1, out_imf.at[od])
            cp1 = pltpu.async_copy(imlp_hbm.at[ii], buf1, sem1)
            cp0.wait()
            pltpu.sync_copy(buf0, out_umlp.at[od])
            cp1.wait()
            pltpu.sync_copy(buf1, out_imlp.at[od])
            return _

        lax.fori_loop(0, nchunk, chunk_body, None)

    return k(quidx, qiidx, p_umf, p_imf, p_umlp, p_imlp)


def _tower_body(gumlp, gimlp, gumf, gimf, uq, iq, w1a, w1b, b1, w2, b2,
                w3, b3, womlp, womf, bo, out):
    def unpack(g, q):
        w = lax.bitcast_convert_type(g[...], jnp.uint32)
        sel = jnp.where(q < 2, w[:, 0:64], w[:, 64:128])
        hb = jnp.where((q & 1) == 1, sel >> 16, sel & 0xFFFF)
        bf = lax.bitcast_convert_type(hb.astype(jnp.uint16), jnp.bfloat16)
        return bf.astype(jnp.float32)

    umlp = unpack(gumlp, uq[...])
    imlp = unpack(gimlp, iq[...])
    umf = unpack(gumf, uq[...])
    imf = unpack(gimf, iq[...])
    h = (jnp.dot(umlp, w1a[...], preferred_element_type=jnp.float32)
         + jnp.dot(imlp, w1b[...], preferred_element_type=jnp.float32)
         + b1[...])
    h = jnp.maximum(h, 0.0)
    h = jnp.maximum(
        jnp.dot(h, w2[...], preferred_element_type=jnp.float32) + b2[...], 0.0)
    h = jnp.maximum(
        jnp.dot(h, w3[...], preferred_element_type=jnp.float32) + b3[...], 0.0)
    mf = umf * imf
    logit = (jnp.sum(h * womlp[...], axis=-1, keepdims=True)
             + jnp.sum(mf * womf[...], axis=-1, keepdims=True)
             + bo[...])
    out[...] = jax.nn.sigmoid(logit)


def _tower(gumlp, gimlp, gumf, gimf, uq, iq,
           W1, b1, W2, b2, W3, b3, W_out, b_out):
    B = gumf.shape[0]
    D = 64
    grid = 8
    bm = B // grid
    w1a, w1b = W1[:D], W1[D:]
    womlp = W_out[:16, 0].reshape(1, 16)
    womf = W_out[16:, 0].reshape(1, D)

    full = lambda i: (0, 0)
    row = lambda i: (i, 0)
    out = pl.pallas_call(
        _tower_body,
        grid=(grid,),
        in_specs=[
            pl.BlockSpec((bm, 2 * D), row),
            pl.BlockSpec((bm, 2 * D), row),
            pl.BlockSpec((bm, 2 * D), row),
            pl.BlockSpec((bm, 2 * D), row),
            pl.BlockSpec((bm, 1), row),
            pl.BlockSpec((bm, 1), row),
            pl.BlockSpec((D, 64), full),
            pl.BlockSpec((D, 64), full),
            pl.BlockSpec((1, 64), full),
            pl.BlockSpec((64, 32), full),
            pl.BlockSpec((1, 32), full),
            pl.BlockSpec((32, 16), full),
            pl.BlockSpec((1, 16), full),
            pl.BlockSpec((1, 16), full),
            pl.BlockSpec((1, D), full),
            pl.BlockSpec((1, 1), full),
        ],
        out_specs=pl.BlockSpec((bm, 1), row),
        out_shape=jax.ShapeDtypeStruct((B, 1), jnp.float32),
        compiler_params=pltpu.CompilerParams(
            dimension_semantics=("arbitrary",)),
    )(gumlp, gimlp, gumf, gimf, uq, iq, w1a, w1b, b1.reshape(1, -1), W2,
      b2.reshape(1, -1), W3, b3.reshape(1, -1), womlp, womf,
      b_out.reshape(1, 1))
    return out[:, 0]


def kernel(user_indices, item_indices, emb_user_mf, emb_item_mf,
           emb_user_mlp, emb_item_mlp, W1, b1, W2, b2, W3, b3, W_out, b_out):
    ui = user_indices.astype(jnp.int32)
    ii = item_indices.astype(jnp.int32)
    p_umf, p_imf, p_umlp, p_imlp = _quad4(
        emb_user_mf.T, emb_item_mf.T, emb_user_mlp.T, emb_item_mlp.T)
    uq = ui // _Q
    iq = ii // _Q
    gumf, gimf, gumlp, gimlp = _gather4(
        ui - uq * _Q, ii - iq * _Q, p_umf, p_imf, p_umlp, p_imlp)
    return _tower(gumlp, gimlp, gumf, gimf,
                  uq.reshape(-1, 1), iq.reshape(-1, 1),
                  W1, b1, W2, b2, W3, b3, W_out, b_out)


# speedup vs baseline: 2.2436x; 1.0190x over previous
import functools

import jax
import jax.numpy as jnp
from jax import lax
from jax.experimental import pallas as pl
from jax.experimental.pallas import tpu as pltpu
from jax.experimental.pallas import tpu_sc as plsc

_NC = 2
_NS = 16
_NW = _NC * _NS
_QW = 4096
_QB = 62
_Q = _QB * _QW
_CHUNK = 256


def _quad_body(*refs):
    ins, outs = refs[:16], refs[16:]
    eye = jnp.eye(64, dtype=jnp.float32)

    def t16(x):
        xt = lax.dot_general(x[...], eye, (((0,), (0,)), ((), ())),
                             preferred_element_type=jnp.float32)
        b = lax.bitcast_convert_type(xt.astype(jnp.bfloat16), jnp.uint16)
        return b.astype(jnp.uint32)

    for i, o in enumerate(outs):
        a0, a1, a2, a3 = (t16(x) for x in ins[4 * i:4 * i + 4])
        o[:, :64] = lax.bitcast_convert_type((a1 << 16) | a0, jnp.float32)
        o[:, 64:] = lax.bitcast_convert_type((a3 << 16) | a2, jnp.float32)


def _quad4(t0, t1, t2, t3):
    D = 64
    nblk = (t0.shape[1] + _QW - 1) // _QW
    ins = []
    specs = []
    for t in (t0, t1, t2, t3):
        for k in range(4):
            ins.append(t)
            specs.append(
                pl.BlockSpec((D, _QW), functools.partial(
                    lambda k, j: (0, jnp.minimum(j + k * _QB, nblk - 1)), k)))
    spec_out = pl.BlockSpec((_QW, 128), lambda j: (j, 0))
    return pl.pallas_call(
        _quad_body,
        grid=(_QB,),
        in_specs=specs,
        out_specs=[spec_out] * 4,
        out_shape=[jax.ShapeDtypeStruct((_Q, 128), jnp.float32)] * 4,
        compiler_params=pltpu.CompilerParams(
            dimension_semantics=("arbitrary",)),
    )(*ins)


def _gather4(quidx, qiidx, p_umf, p_imf, p_umlp, p_imlp):
    B = quidx.shape[0]
    D = p_umf.shape[1]
    bw = B // _NW
    nchunk = bw // _CHUNK

    mesh = plsc.VectorSubcoreMesh(
        core_axis_name="c", subcore_axis_name="s",
        num_cores=_NC, num_subcores=_NS)

    @functools.partial(
        pl.kernel,
        mesh=mesh,
        out_type=[jax.ShapeDtypeStruct((B, D), jnp.float32)] * 4,
        scratch_types=[
            pltpu.VMEM((bw,), jnp.int32),
            pltpu.VMEM((bw,), jnp.int32),
            pltpu.VMEM((_CHUNK, D), jnp.float32),
            pltpu.VMEM((_CHUNK, D), jnp.float32),
            pltpu.SemaphoreType.DMA,
            pltpu.SemaphoreType.DMA,
        ],
    )
    def k(uidx_hbm, iidx_hbm, umf_hbm, imf_hbm, umlp_hbm, imlp_hbm,
          out_umf, out_imf, out_umlp, out_imlp,
          uidx_v, iidx_v, buf0, buf1, sem0, sem1):
        wid = lax.axis_index("s") * _NC + lax.axis_index("c")
        base = wid * bw
        pltpu.sync_copy(uidx_hbm.at[pl.ds(base, bw)], uidx_v)
        pltpu.sync_copy(iidx_hbm.at[pl.ds(base, bw)], iidx_v)

        def chunk_body(c, _):
            cb = c * _CHUNK
            iu = uidx_v.at[pl.ds(cb, _CHUNK)]
            ii = iidx_v.at[pl.ds(cb, _CHUNK)]
            od = pl.ds(base + cb, _CHUNK)
            cp0 = pltpu.async_copy(umf_hbm.at[iu], buf0, sem0)
            cp1 = pltpu.async_copy(imf_hbm.at[ii], buf1, sem1)
            cp0.wait()
            pltpu.sync_copy(buf0, out_umf.at[od])
            cp0 = pltpu.async_copy(umlp_hbm.at[iu], buf0, sem0)
            cp1.wait()
            pltpu.sync_copy(buf1, out_imf.at[od])
            cp1 = pltpu.async_copy(imlp_hbm.at[ii], buf1, sem1)
            cp0.wait()
            pltpu.sync_copy(buf0, out_umlp.at[od])
            cp1.wait()
            pltpu.sync_copy(buf1, out_imlp.at[od])
            return _

        lax.fori_loop(0, nchunk, chunk_body, None)

    return k(quidx, qiidx, p_umf, p_imf, p_umlp, p_imlp)


def _tower_body(gumlp, gimlp, gumf, gimf, uq, iq, w1a, w1b, b1, w2, b2,
                w3, b3, womlp, womf, bo, out):
    def unpack(g, q):
        w = lax.bitcast_convert_type(g[...], jnp.uint32)
        sel = jnp.where(q < 2, w[:, 0:64], w[:, 64:128])
        hb = jnp.where((q & 1) == 1, sel >> 16, sel & 0xFFFF)
        bf = lax.bitcast_convert_type(hb.astype(jnp.uint16), jnp.bfloat16)
        return bf.astype(jnp.float32)

    umlp = unpack(gumlp, uq[...])
    imlp = unpack(gimlp, iq[...])
    umf = unpack(gumf, uq[...])
    imf = unpack(gimf, iq[...])
    h = (jnp.dot(umlp, w1a[...], preferred_element_type=jnp.float32)
         + jnp.dot(imlp, w1b[...], preferred_element_type=jnp.float32)
         + b1[...])
    h = jnp.maximum(h, 0.0)
    h = jnp.maximum(
        jnp.dot(h, w2[...], preferred_element_type=jnp.float32) + b2[...], 0.0)
    h = jnp.maximum(
        jnp.dot(h, w3[...], preferred_element_type=jnp.float32) + b3[...], 0.0)
    mf = umf * imf
    logit = (jnp.sum(h * womlp[...], axis=-1, keepdims=True)
             + jnp.sum(mf * womf[...], axis=-1, keepdims=True)
             + bo[...])
    out[...] = jax.nn.sigmoid(logit)


def _tower(gumlp, gimlp, gumf, gimf, uq, iq,
           W1, b1, W2, b2, W3, b3, W_out, b_out):
    B = gumf.shape[0]
    D = 64
    grid = 8
    bm = B // grid
    w1a, w1b = W1[:D], W1[D:]
    womlp = W_out[:16, 0].reshape(1, 16)
    womf = W_out[16:, 0].reshape(1, D)

    full = lambda i: (0, 0)
    row = lambda i: (i, 0)
    out = pl.pallas_call(
        _tower_body,
        grid=(grid,),
        in_specs=[
            pl.BlockSpec((bm, 2 * D), row),
            pl.BlockSpec((bm, 2 * D), row),
            pl.BlockSpec((bm, 2 * D), row),
            pl.BlockSpec((bm, 2 * D), row),
            pl.BlockSpec((bm, 1), row),
            pl.BlockSpec((bm, 1), row),
            pl.BlockSpec((D, 64), full),
            pl.BlockSpec((D, 64), full),
            pl.BlockSpec((1, 64), full),
            pl.BlockSpec((64, 32), full),
            pl.BlockSpec((1, 32), full),
            pl.BlockSpec((32, 16), full),
            pl.BlockSpec((1, 16), full),
            pl.BlockSpec((1, 16), full),
            pl.BlockSpec((1, D), full),
            pl.BlockSpec((1, 1), full),
        ],
        out_specs=pl.BlockSpec((bm, 1), row),
        out_shape=jax.ShapeDtypeStruct((B, 1), jnp.float32),
        compiler_params=pltpu.CompilerParams(
            dimension_semantics=("arbitrary",)),
    )(gumlp, gimlp, gumf, gimf, uq, iq, w1a, w1b, b1.reshape(1, -1), W2,
      b2.reshape(1, -1), W3, b3.reshape(1, -1), womlp, womf,
      b_out.reshape(1, 1))
    return out[:, 0]


def kernel(user_indices, item_indices, emb_user_mf, emb_item_mf,
           emb_user_mlp, emb_item_mlp, W1, b1, W2, b2, W3, b3, W_out, b_out):
    ui = user_indices.astype(jnp.int32)
    ii = item_indices.astype(jnp.int32)
    p_umf, p_imf, p_umlp, p_imlp = _quad4(
        emb_user_mf.T, emb_item_mf.T, emb_user_mlp.T, emb_item_mlp.T)
    uq = ui // _Q
    iq = ii // _Q
    gumf, gimf, gumlp, gimlp = _gather4(
        ui - uq * _Q, ii - iq * _Q, p_umf, p_imf, p_umlp, p_imlp)
    return _tower(gumlp, gimlp, gumf, gimf,
                  uq.reshape(-1, 1), iq.reshape(-1, 1),
                  W1, b1, W2, b2, W3, b3, W_out, b_out)
